# Initial kernel scaffold; baseline (speedup 1.0000x reference)
import jax
import jax.numpy as jnp
from jax.experimental import pallas as pl


def kernel(embeddings_x, embeddings_y, labels):
    raise NotImplementedError("write your pallas kernel here")


# speedup vs baseline: 2.2111x; 2.2111x over previous
import jax
import jax.numpy as jnp
from jax.experimental import pallas as pl

N = 256
MARGIN = 1.0
BIG = 1e30
CHUNK = 16


def _loss_body(x_ref, y_ref, lab_ref, out_ref):
    x = x_ref[:]
    y = y_ref[:]
    lab = lab_ref[:]

    g = jnp.dot(x, y.T, preferred_element_type=jnp.float32)
    nx = jnp.sum(x * x, axis=1, keepdims=True)
    ny = jnp.sum(y * y, axis=1, keepdims=True)
    dmat = nx + ny.T - 2.0 * g

    same = lab.T == lab
    ii = jax.lax.broadcasted_iota(jnp.int32, (N, N), 0)
    jj = jax.lax.broadcasted_iota(jnp.int32, (N, N), 1)
    pm = same & (ii < jj)
    nm = ~same

    npos = jnp.sum(pm.astype(jnp.float32), axis=1)
    nneg = jnp.sum(nm.astype(jnp.float32), axis=1)
    count = jnp.sum(npos * nneg)

    dmat_t = dmat.T
    ax = jnp.where(pm, dmat + MARGIN, -BIG)
    bx = jnp.where(nm, dmat, BIG)
    ay = jnp.where(pm, dmat_t + MARGIN, -BIG)
    by = jnp.where(nm, dmat_t, BIG)

    total = jnp.float32(0.0)
    for c in range(N // CHUNK):
        lo, hi = c * CHUNK, (c + 1) * CHUNK
        sx = jnp.sum(jax.nn.relu(ax[lo:hi, :, None] - bx[lo:hi, None, :]))
        sy = jnp.sum(jax.nn.relu(ay[lo:hi, :, None] - by[lo:hi, None, :]))
        total = total + sx + sy
    out_ref[:, :] = jnp.broadcast_to(total / count, (1, 1))


def kernel(embeddings_x, embeddings_y, labels):
    out = pl.pallas_call(
        _loss_body,
        out_shape=jax.ShapeDtypeStruct((1, 1), jnp.float32),
    )(embeddings_x, embeddings_y, labels.reshape(1, N))
    return out[0, 0]


# speedup vs baseline: 2.4029x; 1.0867x over previous
import dataclasses

import jax
import jax.numpy as jnp
from jax.experimental import pallas as pl
from jax.experimental.pallas import tpu as pltpu
from jax.experimental.pallas import tpu_sc as plsc

N = 256
MARGIN = 1.0
BIG = 1e30
LANES = 16
NCHUNK = N // LANES
NTILES = 32
APT = N // NTILES // 2
ROWS = 2 * APT


def _dist_body(x_ref, y_ref, d_ref, dt_ref):
    x = x_ref[:]
    y = y_ref[:]
    g = jnp.dot(x, y.T, preferred_element_type=jnp.float32)
    nx = jnp.sum(x * x, axis=1, keepdims=True)
    ny = jnp.sum(y * y, axis=1, keepdims=True)
    d = nx + ny.T - 2.0 * g
    d_ref[:, :] = d
    dt_ref[:, :] = d.T


def _sc_loss_body(d_hbm, dt_hbm, lab_hbm, out_hbm,
                  d_vm, dt_vm, lab_vm, bx_vm, by_vm, stage_vm, sem):
    core = jax.lax.axis_index("c")
    sub = jax.lax.axis_index("s")
    t = core * 16 + sub
    lane = jax.lax.iota(jnp.int32, LANES)

    front0 = t * APT
    back0 = N - APT - t * APT

    cp_lab = pltpu.async_copy(lab_hbm, lab_vm, sem)
    cp_df = pltpu.async_copy(d_hbm.at[pl.ds(front0 * N, APT * N)],
                             d_vm.at[pl.ds(0, APT * N)], sem)
    cp_db = pltpu.async_copy(d_hbm.at[pl.ds(back0 * N, APT * N)],
                             d_vm.at[pl.ds(APT * N, APT * N)], sem)
    cp_tf = pltpu.async_copy(dt_hbm.at[pl.ds(front0 * N, APT * N)],
                             dt_vm.at[pl.ds(0, APT * N)], sem)
    cp_tb = pltpu.async_copy(dt_hbm.at[pl.ds(back0 * N, APT * N)],
                             dt_vm.at[pl.ds(APT * N, APT * N)], sem)
    cp_lab.wait()
    cp_df.wait()
    cp_db.wait()
    cp_tf.wait()
    cp_tb.wait()

    def anchor_body(row, carry):
        accx0, accy0, cnt = carry
        i = jnp.where(row < APT, front0 + row, back0 + (row - APT))
        base = row * N
        lab_i = plsc.load_gather(lab_vm, [jnp.full((LANES,), i, jnp.int32)])

        def mask_body(c, nsame):
            sl = pl.ds(c * LANES, LANES)
            samev = lab_vm[sl] == lab_i
            bx_vm[sl] = jnp.where(samev, BIG, d_vm[pl.ds(base + c * LANES, LANES)])
            by_vm[sl] = jnp.where(samev, BIG, dt_vm[pl.ds(base + c * LANES, LANES)])
            return nsame + jnp.max(plsc.all_reduce_population_count(samev))

        nsame = jax.lax.fori_loop(0, NCHUNK, mask_body, jnp.int32(0))
        nneg = (jnp.int32(N) - nsame).astype(jnp.float32)

        def chunk_body(c, car):
            npos, ax0, ay0 = car
            sl = pl.ds(c * LANES, LANES)
            pmv = (lab_vm[sl] == lab_i) & (lane + c * LANES > i)
            pcnt = jnp.max(plsc.all_reduce_population_count(pmv))
            pmi0 = jnp.where(pmv, jnp.int32(1), jnp.int32(0))

            def pair_body(_, pc):
                pmi, ax1, ay1 = pc
                idx = jnp.max(plsc.all_reduce_ffs(pmi != 0))
                jsp = jnp.full((LANES,), base + idx + c * LANES, jnp.int32)
                a_x = plsc.load_gather(d_vm, [jsp]) + MARGIN
                a_y = plsc.load_gather(dt_vm, [jsp]) + MARGIN

                def k_body(c2, s):
                    sx, sy = s
                    sl2 = pl.ds(c2 * LANES, LANES)
                    sx = sx + jnp.maximum(a_x - bx_vm[sl2], 0.0)
                    sy = sy + jnp.maximum(a_y - by_vm[sl2], 0.0)
                    return sx, sy

                sx, sy = jax.lax.fori_loop(
                    0, NCHUNK, k_body,
                    (jnp.zeros((LANES,), jnp.float32),
                     jnp.zeros((LANES,), jnp.float32)))
                pmi = jnp.where(lane == idx, jnp.int32(0), pmi)
                return pmi, ax1 + sx, ay1 + sy

            _, ax0, ay0 = jax.lax.fori_loop(0, pcnt, pair_body, (pmi0, ax0, ay0))
            return npos + pcnt, ax0, ay0

        npos, accx0, accy0 = jax.lax.fori_loop(
            0, NCHUNK, chunk_body, (jnp.int32(0), accx0, accy0))
        return accx0, accy0, cnt + npos.astype(jnp.float32) * nneg

    accx, accy, cnt = jax.lax.fori_loop(
        0, ROWS, anchor_body,
        (jnp.zeros((LANES,), jnp.float32),
         jnp.zeros((LANES,), jnp.float32),
         jnp.float32(0.0)))

    total = jnp.sum(accx) + jnp.sum(accy)
    stage_vm[:] = (jnp.where(lane == 0, total, 0.0)
                   + jnp.where(lane == 1, cnt, 0.0))
    pltpu.async_copy(stage_vm, out_hbm.at[t], sem).wait()


def kernel(embeddings_x, embeddings_y, labels):
    d, dt = pl.pallas_call(
        _dist_body,
        out_shape=[
            jax.ShapeDtypeStruct((N, N), jnp.float32),
            jax.ShapeDtypeStruct((N, N), jnp.float32),
        ],
    )(embeddings_x, embeddings_y)

    cp = pltpu.CompilerParams()
    if "needs_layout_passes" in pltpu.CompilerParams.__dataclass_fields__:
        cp = dataclasses.replace(cp, needs_layout_passes=False)
    mesh = plsc.VectorSubcoreMesh(core_axis_name="c", subcore_axis_name="s")
    sc_loss = pl.kernel(
        _sc_loss_body,
        out_type=jax.ShapeDtypeStruct((NTILES, LANES), jnp.float32),
        mesh=mesh,
        compiler_params=cp,
        scratch_types=[
            pltpu.VMEM((ROWS * N,), jnp.float32),
            pltpu.VMEM((ROWS * N,), jnp.float32),
            pltpu.VMEM((N,), jnp.int32),
            pltpu.VMEM((N,), jnp.float32),
            pltpu.VMEM((N,), jnp.float32),
            pltpu.VMEM((LANES,), jnp.float32),
            pltpu.SemaphoreType.DMA,
        ],
    )
    partials = sc_loss(d.reshape(-1), dt.reshape(-1), labels)
    total = jnp.sum(partials[:, 0])
    count = jnp.sum(partials[:, 1])
    return total / count
